# Initial kernel scaffold; baseline (speedup 1.0000x reference)
import jax
import jax.numpy as jnp
from jax.experimental import pallas as pl


def kernel(x, gates, choices):
    raise NotImplementedError("write your pallas kernel here")


# speedup vs baseline: 1252.8754x; 1252.8754x over previous
import functools

import jax
import jax.numpy as jnp
from jax import lax
from jax.experimental import pallas as pl
from jax.experimental.pallas import tpu as pltpu
from jax.experimental.pallas import tpu_sc as plsc

NC, NS = 2, 16
NW = NC * NS
LANES = 16

BATCH = 1024
N_INPUTS = 4096
N_GATES = 16384
W = BATCH // 32
GPT = N_GATES // NW
IDX_CH = (2 * GPT) // 128


def _pack_body(x_ref, out_ref):
    xb = x_ref[...]
    j = lax.broadcasted_iota(jnp.int32, xb.shape, 0)
    out_ref[...] = jnp.sum(xb << j, axis=0, keepdims=True)[None]


_pack = pl.pallas_call(
    _pack_body,
    grid=(W,),
    in_specs=[pl.BlockSpec((32, N_INPUTS), lambda i: (i, 0))],
    out_specs=pl.BlockSpec((1, 1, N_INPUTS), lambda i: (i, 0, 0)),
    out_shape=jax.ShapeDtypeStruct((W, 1, N_INPUTS), jnp.int32),
)


def _unpack_body(p_ref, out_ref):
    row = p_ref[0]
    j = lax.broadcasted_iota(jnp.int32, (32, N_GATES), 0)
    bits = jnp.broadcast_to(row, (32, N_GATES))
    out_ref[...] = ((bits >> j) & 1) != 0


_unpack = pl.pallas_call(
    _unpack_body,
    grid=(W,),
    in_specs=[pl.BlockSpec((1, 1, N_GATES), lambda i: (i, 0, 0))],
    out_specs=pl.BlockSpec((32, N_GATES), lambda i: (i, 0)),
    out_shape=jax.ShapeDtypeStruct((BATCH, N_GATES), jnp.bool_),
)


@functools.cache
def _make_sc_gate():
    mesh = plsc.VectorSubcoreMesh(
        core_axis_name="c", subcore_axis_name="s", num_cores=NC, num_subcores=NS)
    return functools.partial(
        pl.kernel,
        out_type=jax.ShapeDtypeStruct((N_GATES, W), jnp.int32),
        mesh=mesh,
        compiler_params=pltpu.CompilerParams(use_tc_tiling_on_sc=False),
        scratch_types=[
            pltpu.VMEM((IDX_CH, 128), jnp.int32),
            pltpu.VMEM((2 * GPT, W), jnp.int32),
            pltpu.VMEM((GPT,), jnp.int32),
            pltpu.VMEM((GPT, W), jnp.int32),
            pltpu.SemaphoreType.DMA,
        ],
    )(_sc_gate_body)


def _sc_gate_body(xtp_hbm, cidx_hbm, tt_hbm, out_hbm, idx_v, rows_v, tt_v, out_v, sem):
    wid = lax.axis_index("s") * NC + lax.axis_index("c")
    base = wid * GPT
    pltpu.sync_copy(cidx_hbm.at[wid], idx_v)
    pltpu.sync_copy(tt_hbm.at[pl.ds(base, GPT)], tt_v)
    descs = [
        pltpu.async_copy(xtp_hbm.at[idx_v.at[j]],
                         rows_v.at[pl.ds(j * 128, 128)], sem)
        for j in range(IDX_CH)
    ]
    for d in descs:
        d.wait()

    def body(q, _):
        tvec = tt_v[pl.ds(q * LANES, LANES)]
        for i in range(LANES):
            g = q * LANES + i
            tb = tvec[i]
            t0 = -(tb & 1)
            t1 = -((tb >> 1) & 1)
            t2 = -((tb >> 2) & 1)
            t3 = -((tb >> 3) & 1)
            for h in range(W // LANES):
                av = rows_v[2 * g, pl.ds(h * LANES, LANES)]
                bv = rows_v[2 * g + 1, pl.ds(h * LANES, LANES)]
                out_v[g, pl.ds(h * LANES, LANES)] = (
                    t0 ^ (av & t2) ^ (bv & t1) ^ (av & bv & t3))
        return 0

    lax.fori_loop(0, GPT // LANES, body, 0)
    pltpu.sync_copy(out_v, out_hbm.at[pl.ds(base, GPT)])


def kernel(x, gates, choices):
    xp = _pack(x).reshape(W, N_INPUTS)
    xtp = jnp.transpose(xp)
    g = gates.astype(jnp.int32)
    t0 = g[:, 0]
    t1 = g[:, 0] ^ g[:, 1]
    t2 = g[:, 0] ^ g[:, 2]
    t3 = g[:, 0] ^ g[:, 1] ^ g[:, 2] ^ g[:, 3]
    tt = t0 | (t1 << 1) | (t2 << 2) | (t3 << 3)
    cidx = choices.reshape(NW, IDX_CH, 128)
    pkt = _make_sc_gate()(xtp, cidx, tt)
    return _unpack(jnp.transpose(pkt).reshape(W, 1, N_GATES))


# speedup vs baseline: 1683.0740x; 1.3434x over previous
import functools

import jax
import jax.numpy as jnp
from jax import lax
from jax.experimental import pallas as pl
from jax.experimental.pallas import tpu as pltpu
from jax.experimental.pallas import tpu_sc as plsc

NC, NS = 2, 16
NW = NC * NS
LANES = 16

BATCH = 1024
N_INPUTS = 4096
N_GATES = 16384
W = BATCH // 32
GPT = N_GATES // NW
IDX_CH = (2 * GPT) // 128


PACK_CB = 512
UNPACK_GB = 512


def _pack_body(x_ref, out_ref):
    xb = x_ref[...]
    x3 = xb.reshape(32, 32, PACK_CB)
    j = lax.broadcasted_iota(jnp.int32, x3.shape, 1)
    p = jnp.sum(x3 << j, axis=1)
    out_ref[...] = jnp.transpose(p)


_pack = pl.pallas_call(
    _pack_body,
    grid=(N_INPUTS // PACK_CB,),
    in_specs=[pl.BlockSpec((BATCH, PACK_CB), lambda i: (0, i))],
    out_specs=pl.BlockSpec((PACK_CB, W), lambda i: (i, 0)),
    out_shape=jax.ShapeDtypeStruct((N_INPUTS, W), jnp.int32),
)


def _unpack_body(p_ref, out_ref):
    pk = p_ref[0]
    t = jnp.transpose(pk)
    b3 = jnp.broadcast_to(t[:, None, :], (W, 32, UNPACK_GB))
    j = lax.broadcasted_iota(jnp.int32, b3.shape, 1)
    out_ref[...] = (((b3 >> j) & 1).astype(jnp.int8)).reshape(BATCH, UNPACK_GB)


_unpack = pl.pallas_call(
    _unpack_body,
    grid=(N_GATES // UNPACK_GB,),
    in_specs=[pl.BlockSpec((1, UNPACK_GB, W), lambda i: (i, 0, 0))],
    out_specs=pl.BlockSpec((BATCH, UNPACK_GB), lambda i: (0, i)),
    out_shape=jax.ShapeDtypeStruct((BATCH, N_GATES), jnp.int8),
)


@functools.cache
def _make_sc_gate():
    mesh = plsc.VectorSubcoreMesh(
        core_axis_name="c", subcore_axis_name="s", num_cores=NC, num_subcores=NS)
    return functools.partial(
        pl.kernel,
        out_type=jax.ShapeDtypeStruct((N_GATES, W), jnp.int32),
        mesh=mesh,
        compiler_params=pltpu.CompilerParams(use_tc_tiling_on_sc=False),
        scratch_types=[
            pltpu.VMEM((IDX_CH, 128), jnp.int32),
            pltpu.VMEM((2 * GPT, W), jnp.int32),
            pltpu.VMEM((GPT,), jnp.int32),
            pltpu.VMEM((GPT, W), jnp.int32),
            pltpu.SemaphoreType.DMA,
        ],
    )(_sc_gate_body)


def _sc_gate_body(xtp_hbm, cidx_hbm, tt_hbm, out_hbm, idx_v, rows_v, tt_v, out_v, sem):
    wid = lax.axis_index("s") * NC + lax.axis_index("c")
    base = wid * GPT
    pltpu.sync_copy(cidx_hbm.at[wid], idx_v)
    pltpu.sync_copy(tt_hbm.at[pl.ds(base, GPT)], tt_v)
    descs = [
        pltpu.async_copy(xtp_hbm.at[idx_v.at[j]],
                         rows_v.at[pl.ds(j * 128, 128)], sem)
        for j in range(IDX_CH)
    ]
    for d in descs:
        d.wait()

    def body(q, _):
        tvec = tt_v[pl.ds(q * LANES, LANES)]
        for i in range(LANES):
            g = q * LANES + i
            tb = tvec[i]
            t0 = -(tb & 1)
            t1 = -((tb >> 1) & 1)
            t2 = -((tb >> 2) & 1)
            t3 = -((tb >> 3) & 1)
            for h in range(W // LANES):
                av = rows_v[2 * g, pl.ds(h * LANES, LANES)]
                bv = rows_v[2 * g + 1, pl.ds(h * LANES, LANES)]
                out_v[g, pl.ds(h * LANES, LANES)] = (
                    t0 ^ (av & t2) ^ (bv & t1) ^ (av & bv & t3))
        return 0

    lax.fori_loop(0, GPT // LANES, body, 0)
    pltpu.sync_copy(out_v, out_hbm.at[pl.ds(base, GPT)])


def kernel(x, gates, choices):
    xtp = _pack(x)
    g = gates.astype(jnp.int32)
    t0 = g[:, 0]
    t1 = g[:, 0] ^ g[:, 1]
    t2 = g[:, 0] ^ g[:, 2]
    t3 = g[:, 0] ^ g[:, 1] ^ g[:, 2] ^ g[:, 3]
    tt = t0 | (t1 << 1) | (t2 << 2) | (t3 << 3)
    cidx = choices.reshape(NW, IDX_CH, 128)
    pkt = _make_sc_gate()(xtp, cidx, tt)
    out8 = _unpack(pkt.reshape(N_GATES // UNPACK_GB, UNPACK_GB, W))
    return out8.view(jnp.bool_)


# speedup vs baseline: 1802.2957x; 1.0708x over previous
import functools

import jax
import jax.numpy as jnp
from jax import lax
from jax.experimental import pallas as pl
from jax.experimental.pallas import tpu as pltpu
from jax.experimental.pallas import tpu_sc as plsc

NC, NS = 2, 16
NW = NC * NS
LANES = 16

BATCH = 1024
N_INPUTS = 4096
N_GATES = 16384
W = BATCH // 32
GPT = N_GATES // NW
IDX_CH = (2 * GPT) // 128


PACK_CB = 512
UNPACK_GB = 1024


def _pack_body(x_ref, out_ref):
    xb = x_ref[...]
    x3 = xb.reshape(32, 32, PACK_CB)
    j = lax.broadcasted_iota(jnp.int32, x3.shape, 1)
    p = jnp.sum(x3 << j, axis=1)
    out_ref[...] = jnp.transpose(p)


_pack = pl.pallas_call(
    _pack_body,
    grid=(N_INPUTS // PACK_CB,),
    in_specs=[pl.BlockSpec((BATCH, PACK_CB), lambda i: (0, i))],
    out_specs=pl.BlockSpec((PACK_CB, W), lambda i: (i, 0)),
    out_shape=jax.ShapeDtypeStruct((N_INPUTS, W), jnp.int32),
)


def _unpack_body(p_ref, out_ref):
    pk = p_ref[...]
    t = jnp.transpose(pk)
    b3 = jnp.broadcast_to(t[:, None, :], (W, 32, UNPACK_GB))
    j = lax.broadcasted_iota(jnp.int32, b3.shape, 1)
    out_ref[...] = (((b3 >> j) & 1).astype(jnp.int8)).reshape(BATCH, UNPACK_GB)


_unpack = pl.pallas_call(
    _unpack_body,
    grid=(N_GATES // UNPACK_GB,),
    in_specs=[pl.BlockSpec((UNPACK_GB, W), lambda i: (i, 0))],
    out_specs=pl.BlockSpec((BATCH, UNPACK_GB), lambda i: (0, i)),
    out_shape=jax.ShapeDtypeStruct((BATCH, N_GATES), jnp.int8),
)


@functools.cache
def _make_sc_gate():
    mesh = plsc.VectorSubcoreMesh(
        core_axis_name="c", subcore_axis_name="s", num_cores=NC, num_subcores=NS)
    return functools.partial(
        pl.kernel,
        out_type=jax.ShapeDtypeStruct((N_GATES, W), jnp.int32),
        mesh=mesh,
        compiler_params=pltpu.CompilerParams(use_tc_tiling_on_sc=False),
        scratch_types=[
            pltpu.VMEM((IDX_CH, 128), jnp.int32),
            pltpu.VMEM((2 * GPT, W), jnp.int32),
            pltpu.VMEM((GPT,), jnp.int32),
            pltpu.VMEM((GPT, W), jnp.int32),
            pltpu.SemaphoreType.DMA,
        ],
    )(_sc_gate_body)


def _sc_gate_body(xtp_hbm, cidx_hbm, tt_hbm, out_hbm, idx_v, rows_v, tt_v, out_v, sem):
    wid = lax.axis_index("s") * NC + lax.axis_index("c")
    base = wid * GPT
    pltpu.sync_copy(cidx_hbm.at[wid], idx_v)
    pltpu.sync_copy(tt_hbm.at[pl.ds(base, GPT)], tt_v)
    descs = [
        pltpu.async_copy(xtp_hbm.at[idx_v.at[j]],
                         rows_v.at[pl.ds(j * 128, 128)], sem)
        for j in range(IDX_CH)
    ]
    for d in descs:
        d.wait()

    def body(q, _):
        tvec = tt_v[pl.ds(q * LANES, LANES)]
        for i in range(LANES):
            g = q * LANES + i
            tb = tvec[i]
            t0 = -(tb & 1)
            t1 = -((tb >> 1) & 1)
            t2 = -((tb >> 2) & 1)
            t3 = -((tb >> 3) & 1)
            for h in range(W // LANES):
                av = rows_v[2 * g, pl.ds(h * LANES, LANES)]
                bv = rows_v[2 * g + 1, pl.ds(h * LANES, LANES)]
                out_v[g, pl.ds(h * LANES, LANES)] = (
                    t0 ^ (av & t2) ^ (bv & t1) ^ (av & bv & t3))
        return 0

    lax.fori_loop(0, GPT // LANES, body, 0)
    pltpu.sync_copy(out_v, out_hbm.at[pl.ds(base, GPT)])


def kernel(x, gates, choices):
    xtp = _pack(x)
    g = gates.astype(jnp.int32)
    t0 = g[:, 0]
    t1 = g[:, 0] ^ g[:, 1]
    t2 = g[:, 0] ^ g[:, 2]
    t3 = g[:, 0] ^ g[:, 1] ^ g[:, 2] ^ g[:, 3]
    tt = t0 | (t1 << 1) | (t2 << 2) | (t3 << 3)
    cidx = choices.reshape(NW, IDX_CH, 128)
    pkt = _make_sc_gate()(xtp, cidx, tt)
    out8 = _unpack(pkt)
    return out8.view(jnp.bool_)
